# Initial kernel scaffold; baseline (speedup 1.0000x reference)
import jax
import jax.numpy as jnp
from jax.experimental import pallas as pl


def kernel(idx, emb, targets):
    raise NotImplementedError("write your pallas kernel here")


# speedup vs baseline: 4.7327x; 4.7327x over previous
import functools

import jax
import jax.numpy as jnp
from jax.experimental import pallas as pl
from jax.experimental.pallas import tpu as pltpu


def _fused_tile(tok_ref, tgt_ref, emb_ref, logits_ref, part_ref, *, n_rows):
    tr, v = logits_ref.shape
    tok = tok_ref[0]
    lane = jax.lax.broadcasted_iota(jnp.int32, (tr, v), 1)
    sel = (lane == tok).astype(jnp.bfloat16)
    x = jnp.dot(sel, emb_ref[...],
                preferred_element_type=jnp.float32)
    logits_ref[...] = x

    m = jnp.max(x, axis=-1, keepdims=True)
    lse = jnp.log(jnp.sum(jnp.exp(x - m), axis=-1, keepdims=True)) + m
    tgt = tgt_ref[0]
    picked = jnp.sum(jnp.where(lane == tgt, x, 0.0),
                     axis=-1, keepdims=True)
    per_row = lse - picked

    row0 = pl.program_id(0) * tr
    live = (row0 + jax.lax.broadcasted_iota(jnp.int32, (tr, 1), 0)) < n_rows
    tile_sum = jnp.sum(jnp.where(live, per_row, 0.0))
    part_ref[...] = jnp.full(part_ref.shape, tile_sum, jnp.float32)


def kernel(idx, emb, targets, *, row_tile=512):
    B, T = idx.shape
    V = emb.shape[0]
    N = B * T
    assert V % 128 == 0, "vocab assumed lane-aligned"

    tr = min(row_tile, N)
    n_tiles = -(-N // tr)
    Np = n_tiles * tr

    tok = idx.reshape(N).astype(jnp.int32)
    tgt = targets.reshape(N).astype(jnp.int32)
    if Np != N:
        tok = jnp.pad(tok, (0, Np - N))
        tgt = jnp.pad(tgt, (0, Np - N))
    tok3 = tok.reshape(n_tiles, tr, 1)
    tgt3 = tgt.reshape(n_tiles, tr, 1)
    emb_bf = emb.astype(jnp.bfloat16)

    logits, parts = pl.pallas_call(
        functools.partial(_fused_tile, n_rows=N),
        out_shape=(jax.ShapeDtypeStruct((Np, V), jnp.float32),
                   jax.ShapeDtypeStruct((n_tiles, 8, 128), jnp.float32)),
        grid=(n_tiles,),
        in_specs=[pl.BlockSpec((1, tr, 1), lambda i: (i, 0, 0)),
                  pl.BlockSpec((1, tr, 1), lambda i: (i, 0, 0)),
                  pl.BlockSpec((V, V), lambda i: (0, 0))],
        out_specs=(pl.BlockSpec((tr, V), lambda i: (i, 0)),
                   pl.BlockSpec((1, 8, 128), lambda i: (i, 0, 0))),
        compiler_params=pltpu.CompilerParams(
            dimension_semantics=("parallel",),
            vmem_limit_bytes=48 * 1024 * 1024),
    )(tok3, tgt3, emb_bf)

    loss = jnp.sum(parts[:, 0, 0]) / N
    return logits[:N], loss
